# Initial kernel scaffold; baseline (speedup 1.0000x reference)
import functools

import jax
import jax.numpy as jnp
from jax import lax
from jax.experimental import pallas as pl
from jax.experimental.pallas import tpu as pltpu
from jax.experimental.pallas import tpu_sc as plsc

N = 10000
E = 320000
D = 128
DH = D // 2

NC = 2
NS = 16

EPT = E // NS
C = 400
NCH = EPT // C
RPT = N // NS
ZR = 25
FB = 125


def _sc_body(inp2_hbm, dst_hbm, src_hbm, val_hbm, x_hbm, ab_hbm, out_hbm,
             acc_sh, src_v, dst_v, val_v, rows_v, zbuf, obuf, xbuf, ab_v, sem):
    c = lax.axis_index("c")
    s = lax.axis_index("s")

    zeros16 = jnp.zeros((16,), jnp.float32)
    for r in range(ZR):
        for g in range(DH // 16):
            zbuf[r, pl.ds(g * 16, 16)] = zeros16

    def zero_blk(b, carry):
        pltpu.sync_copy(zbuf, acc_sh.at[pl.ds(s * RPT + b * ZR, ZR)])
        return carry

    lax.fori_loop(0, RPT // ZR, zero_blk, 0)
    plsc.subcore_barrier()

    def chunk_body(k, carry):
        base = s * EPT + k * C
        pltpu.sync_copy(dst_hbm.at[pl.ds(base, C)], dst_v)
        pltpu.sync_copy(src_hbm.at[pl.ds(base, C)], src_v)
        pltpu.sync_copy(val_hbm.at[pl.ds(base, C)], val_v)

        def idx_body(g, carry2):
            v = src_v[pl.ds(g * 16, 16)]
            src_v[pl.ds(g * 16, 16)] = v * 2 + c
            return carry2

        lax.fori_loop(0, C // 16, idx_body, 0)

        pltpu.async_copy(inp2_hbm.at[src_v], rows_v, sem).wait()

        def scale_body(e, carry2):
            vs = plsc.load_gather(val_v, [jnp.full((16,), e, jnp.int32)])
            for g in range(DH // 16):
                rows_v[e, pl.ds(g * 16, 16)] = rows_v[e, pl.ds(g * 16, 16)] * vs
            return carry2

        lax.fori_loop(0, C, scale_body, 0)

        pltpu.sync_copy(rows_v, acc_sh.at[dst_v], add=True)
        return carry

    lax.fori_loop(0, NCH, chunk_body, 0)
    plsc.subcore_barrier()

    pltpu.sync_copy(ab_hbm, ab_v)
    a1 = plsc.load_gather(ab_v, [jnp.zeros((16,), jnp.int32)])
    a2 = plsc.load_gather(ab_v, [jnp.ones((16,), jnp.int32)])

    def fin_blk(b, carry):
        r0 = s * RPT + b * FB
        pltpu.sync_copy(acc_sh.at[pl.ds(r0, FB)], obuf)
        pltpu.sync_copy(x_hbm.at[pl.ds(r0, FB), pl.ds(c * DH, DH)], xbuf)

        def fin_row(r, carry2):
            for g in range(DH // 16):
                ov = obuf[r, pl.ds(g * 16, 16)]
                xv = xbuf[r, pl.ds(g * 16, 16)]
                obuf[r, pl.ds(g * 16, 16)] = a1 * ov + a2 * xv
            return carry2

        lax.fori_loop(0, FB, fin_row, 0)
        pltpu.sync_copy(obuf, out_hbm.at[pl.ds(r0, FB), pl.ds(c * DH, DH)])
        return carry

    lax.fori_loop(0, RPT // FB, fin_blk, 0)


@jax.jit
def _sc_call(inp2, dst, src, val, x, ab):
    mesh = plsc.VectorSubcoreMesh(core_axis_name="c", subcore_axis_name="s")
    f = functools.partial(
        pl.kernel,
        out_type=jax.ShapeDtypeStruct((N, D), jnp.float32),
        mesh=mesh,
        scratch_types=[
            pltpu.VMEM_SHARED((N, DH), jnp.float32),
            pltpu.VMEM((C,), jnp.int32),
            pltpu.VMEM((C,), jnp.int32),
            pltpu.VMEM((C,), jnp.float32),
            pltpu.VMEM((C, DH), jnp.float32),
            pltpu.VMEM((ZR, DH), jnp.float32),
            pltpu.VMEM((FB, DH), jnp.float32),
            pltpu.VMEM((FB, DH), jnp.float32),
            pltpu.VMEM((16,), jnp.float32),
            pltpu.SemaphoreType.DMA,
        ],
    )(_sc_body)
    return f(inp2, dst, src, val, x, ab)


def kernel(inp, adj_indices, adj_values, x, alpha1, alpha2):
    inp2 = inp.reshape(2 * N, DH)
    dst = adj_indices[0]
    src = adj_indices[1]
    ab = jnp.zeros((16,), jnp.float32).at[0].set(alpha1[0]).at[1].set(alpha2[0])
    return _sc_call(inp2, dst, src, adj_values, x, ab)


# speedup vs baseline: 2.5843x; 2.5843x over previous
import functools

import jax
import jax.numpy as jnp
from jax import lax
from jax.experimental import pallas as pl
from jax.experimental.pallas import tpu as pltpu
from jax.experimental.pallas import tpu_sc as plsc

N = 10000
E = 320000
D = 128
DH = D // 2

NC = 2
NS = 16

EPT = E // NS
C = 80
NCH = EPT // C
RPT = N // NS
ZR = 25
FB = 125


def _sc_body(inp2_hbm, dst_hbm, src_hbm, val_hbm, x_hbm, ab_hbm, out_hbm,
             acc_sh, src_v, dst_v, val_v, rows_v, zbuf, obuf, xbuf, ab_v, sem):
    c = lax.axis_index("c")
    s = lax.axis_index("s")

    zeros16 = jnp.zeros((16,), jnp.float32)
    for r in range(ZR):
        for g in range(DH // 16):
            zbuf[r, pl.ds(g * 16, 16)] = zeros16

    def zero_blk(b, carry):
        pltpu.sync_copy(zbuf, acc_sh.at[pl.ds(s * RPT + b * ZR, ZR)])
        return carry

    lax.fori_loop(0, RPT // ZR, zero_blk, 0)
    plsc.subcore_barrier()

    def chunk_body(k, carry):
        base = s * EPT + k * C
        pltpu.sync_copy(dst_hbm.at[pl.ds(base, C)], dst_v)
        pltpu.sync_copy(src_hbm.at[pl.ds(base, C)], src_v)
        pltpu.sync_copy(val_hbm.at[pl.ds(base, C)], val_v)

        def idx_body(g, carry2):
            v = src_v[pl.ds(g * 16, 16)]
            src_v[pl.ds(g * 16, 16)] = v * 2 + c
            return carry2

        lax.fori_loop(0, C // 16, idx_body, 0)

        pltpu.async_copy(inp2_hbm.at[src_v], rows_v, sem).wait()

        def scale_body(e, carry2):
            vs = plsc.load_gather(val_v, [jnp.full((16,), e, jnp.int32)])
            for g in range(DH // 16):
                rows_v[e, pl.ds(g * 16, 16)] = rows_v[e, pl.ds(g * 16, 16)] * vs
            return carry2

        lax.fori_loop(0, C, scale_body, 0)

        pltpu.sync_copy(rows_v, acc_sh.at[dst_v], add=True)
        return carry

    lax.fori_loop(0, NCH, chunk_body, 0)
    plsc.subcore_barrier()

    pltpu.sync_copy(ab_hbm, ab_v)
    a1 = ab_v[0]
    a2 = ab_v[1]

    def fin_blk(b, carry):
        r0 = s * RPT + b * FB
        pltpu.sync_copy(acc_sh.at[pl.ds(r0, FB)], obuf)
        pltpu.sync_copy(x_hbm.at[pl.ds(r0, FB), pl.ds(c * DH, DH)], xbuf)

        def fin_row(r, carry2):
            for g in range(DH // 16):
                ov = obuf[r, pl.ds(g * 16, 16)]
                xv = xbuf[r, pl.ds(g * 16, 16)]
                obuf[r, pl.ds(g * 16, 16)] = a1 * ov + a2 * xv
            return carry2

        lax.fori_loop(0, FB, fin_row, 0)
        pltpu.sync_copy(obuf, out_hbm.at[pl.ds(r0, FB), pl.ds(c * DH, DH)])
        return carry

    lax.fori_loop(0, RPT // FB, fin_blk, 0)


@jax.jit
def _sc_call(inp2, dst, src, val, x, ab):
    mesh = plsc.VectorSubcoreMesh(core_axis_name="c", subcore_axis_name="s")
    f = functools.partial(
        pl.kernel,
        out_type=jax.ShapeDtypeStruct((N, D), jnp.float32),
        mesh=mesh,
        compiler_params=pltpu.CompilerParams(
            use_tc_tiling_on_sc=False, needs_layout_passes=False),
        scratch_types=[
            pltpu.VMEM_SHARED((N, DH), jnp.float32),
            pltpu.VMEM((C,), jnp.int32),
            pltpu.VMEM((C,), jnp.int32),
            pltpu.VMEM((C,), jnp.float32),
            pltpu.VMEM((C, DH), jnp.float32),
            pltpu.VMEM((ZR, DH), jnp.float32),
            pltpu.VMEM((FB, DH), jnp.float32),
            pltpu.VMEM((FB, DH), jnp.float32),
            pltpu.VMEM((2, 16), jnp.float32),
            pltpu.SemaphoreType.DMA,
        ],
    )(_sc_body)
    return f(inp2, dst, src, val, x, ab)


def kernel(inp, adj_indices, adj_values, x, alpha1, alpha2):
    inp2 = inp.reshape(2 * N, DH)
    dst = adj_indices[0]
    src = adj_indices[1]
    ab = jnp.stack([jnp.full((16,), alpha1[0], jnp.float32),
                    jnp.full((16,), alpha2[0], jnp.float32)])
    return _sc_call(inp2, dst, src, adj_values, x, ab)


# speedup vs baseline: 4.8485x; 1.8762x over previous
import functools

import jax
import jax.numpy as jnp
from jax import lax
from jax.experimental import pallas as pl
from jax.experimental.pallas import tpu as pltpu
from jax.experimental.pallas import tpu_sc as plsc

N = 10000
E = 320000
D = 128
DH = D // 2

NC = 2
NS = 16

EPT = E // NS
C = 400
NCH = EPT // C
RPT = N // NS
ZR = 25
FB = 125


def _sc_body(inp2_hbm, dst_hbm, src_hbm, val_hbm, x_hbm, ab_hbm, out_hbm,
             acc_sh, src_v, dst_v, val_v, rows_v, zbuf, obuf, xbuf, ab_v, sem):
    c = lax.axis_index("c")
    s = lax.axis_index("s")

    zeros16 = jnp.zeros((16,), jnp.float32)
    for r in range(ZR):
        for g in range(DH // 16):
            zbuf[r, pl.ds(g * 16, 16)] = zeros16

    def zero_blk(b, carry):
        pltpu.sync_copy(zbuf, acc_sh.at[pl.ds(s * RPT + b * ZR, ZR)])
        return carry

    lax.fori_loop(0, RPT // ZR, zero_blk, 0)
    plsc.subcore_barrier()

    def chunk_body(k, carry):
        base = s * EPT + k * C
        pltpu.sync_copy(dst_hbm.at[pl.ds(base, C)], dst_v)
        pltpu.sync_copy(src_hbm.at[pl.ds(base, C)], src_v)
        pltpu.sync_copy(val_hbm.at[pl.ds(base, C)], val_v)

        def idx_body(g, carry2):
            v = src_v[pl.ds(g * 16, 16)]
            src_v[pl.ds(g * 16, 16)] = v * 2 + c
            return carry2

        lax.fori_loop(0, C // 16, idx_body, 0)

        pltpu.async_copy(inp2_hbm.at[src_v], rows_v, sem).wait()

        def scale_body(e, carry2):
            vs = plsc.load_gather(val_v, [jnp.full((16,), e, jnp.int32)])
            for g in range(DH // 16):
                rows_v[e, pl.ds(g * 16, 16)] = rows_v[e, pl.ds(g * 16, 16)] * vs
            return carry2

        lax.fori_loop(0, C, scale_body, 0)

        pltpu.sync_copy(rows_v, acc_sh.at[dst_v], add=True)
        return carry

    lax.fori_loop(0, NCH, chunk_body, 0)
    plsc.subcore_barrier()

    pltpu.sync_copy(ab_hbm, ab_v)
    a1 = ab_v[0]
    a2 = ab_v[1]

    def fin_blk(b, carry):
        r0 = s * RPT + b * FB
        pltpu.sync_copy(acc_sh.at[pl.ds(r0, FB)], obuf)
        pltpu.sync_copy(x_hbm.at[pl.ds(r0, FB), pl.ds(c * DH, DH)], xbuf)

        def fin_row(r, carry2):
            for g in range(DH // 16):
                ov = obuf[r, pl.ds(g * 16, 16)]
                xv = xbuf[r, pl.ds(g * 16, 16)]
                obuf[r, pl.ds(g * 16, 16)] = a1 * ov + a2 * xv
            return carry2

        lax.fori_loop(0, FB, fin_row, 0)
        pltpu.sync_copy(obuf, out_hbm.at[pl.ds(r0, FB), pl.ds(c * DH, DH)])
        return carry

    lax.fori_loop(0, RPT // FB, fin_blk, 0)


@jax.jit
def _sc_call(inp2, dst, src, val, x, ab):
    mesh = plsc.VectorSubcoreMesh(core_axis_name="c", subcore_axis_name="s")
    f = functools.partial(
        pl.kernel,
        out_type=jax.ShapeDtypeStruct((N, D), jnp.float32),
        mesh=mesh,
        compiler_params=pltpu.CompilerParams(
            use_tc_tiling_on_sc=False, needs_layout_passes=False),
        scratch_types=[
            pltpu.VMEM_SHARED((N, DH), jnp.float32),
            pltpu.VMEM((C,), jnp.int32),
            pltpu.VMEM((C,), jnp.int32),
            pltpu.VMEM((C,), jnp.float32),
            pltpu.VMEM((C, DH), jnp.float32),
            pltpu.VMEM((ZR, DH), jnp.float32),
            pltpu.VMEM((FB, DH), jnp.float32),
            pltpu.VMEM((FB, DH), jnp.float32),
            pltpu.VMEM((2, 16), jnp.float32),
            pltpu.SemaphoreType.DMA,
        ],
    )(_sc_body)
    return f(inp2, dst, src, val, x, ab)


def kernel(inp, adj_indices, adj_values, x, alpha1, alpha2):
    inp2 = inp.reshape(2 * N, DH)
    dst = adj_indices[0]
    src = adj_indices[1]
    ab = jnp.stack([jnp.full((16,), alpha1[0], jnp.float32),
                    jnp.full((16,), alpha2[0], jnp.float32)])
    return _sc_call(inp2, dst, src, adj_values, x, ab)


# speedup vs baseline: 9.3359x; 1.9255x over previous
import functools

import jax
import jax.numpy as jnp
from jax import lax
from jax.experimental import pallas as pl
from jax.experimental.pallas import tpu as pltpu
from jax.experimental.pallas import tpu_sc as plsc

N = 10000
E = 320000
D = 128
DH = D // 2

NC = 2
NS = 16

EPT = E // NS
C = 400
NCH = EPT // C
RPT = N // NS
ZR = 25
FB = 125


def _sc_body(inp2_hbm, dst_hbm, src_hbm, val_hbm, x_hbm, ab_hbm, out_hbm,
             acc_sh, src_a, src_b, dst_a, dst_b, val_a, val_b, rows_a, rows_b,
             zbuf, obuf, xbuf, ab_v, semg_a, semg_b, sems_a, sems_b):
    c = lax.axis_index("c")
    s = lax.axis_index("s")

    zeros16 = jnp.zeros((16,), jnp.float32)
    for r in range(ZR):
        for g in range(DH // 16):
            zbuf[r, pl.ds(g * 16, 16)] = zeros16

    def zero_blk(b, carry):
        pltpu.sync_copy(zbuf, acc_sh.at[pl.ds(s * RPT + b * ZR, ZR)])
        return carry

    lax.fori_loop(0, RPT // ZR, zero_blk, 0)
    plsc.subcore_barrier()

    def load_idx(k, sbuf, dbuf, vbuf):
        base = s * EPT + k * C
        pltpu.sync_copy(dst_hbm.at[pl.ds(base, C)], dbuf)
        pltpu.sync_copy(src_hbm.at[pl.ds(base, C)], sbuf)
        pltpu.sync_copy(val_hbm.at[pl.ds(base, C)], vbuf)

        @plsc.parallel_loop(0, C // 16, unroll=4)
        def _(g):
            v = sbuf[pl.ds(g * 16, 16)]
            sbuf[pl.ds(g * 16, 16)] = v * 2 + c

    def start_gather(sbuf, rbuf, sem):
        pltpu.async_copy(inp2_hbm.at[sbuf], rbuf, sem)

    def wait_gather(sbuf, rbuf, sem):
        pltpu.make_async_copy(inp2_hbm.at[sbuf], rbuf, sem).wait()

    def scale(rbuf, vbuf):
        @plsc.parallel_loop(0, C, unroll=8)
        def _(e):
            vs = plsc.load_gather(vbuf, [jnp.full((16,), e, jnp.int32)])
            for g in range(DH // 16):
                rbuf[e, pl.ds(g * 16, 16)] = rbuf[e, pl.ds(g * 16, 16)] * vs

    def start_scatter(rbuf, dbuf, sem):
        pltpu.async_copy(rbuf, acc_sh.at[dbuf], sem, add=True)

    def wait_scatter(rbuf, dbuf, sem):
        pltpu.make_async_copy(rbuf, acc_sh.at[dbuf], sem).wait()

    load_idx(0, src_a, dst_a, val_a)
    start_gather(src_a, rows_a, semg_a)
    load_idx(1, src_b, dst_b, val_b)
    start_gather(src_b, rows_b, semg_b)

    def pair_body(g, carry):
        k0 = 2 * g
        wait_gather(src_a, rows_a, semg_a)
        scale(rows_a, val_a)
        start_scatter(rows_a, dst_a, sems_a)
        wait_gather(src_b, rows_b, semg_b)
        scale(rows_b, val_b)
        start_scatter(rows_b, dst_b, sems_b)
        wait_scatter(rows_a, dst_a, sems_a)
        load_idx(k0 + 2, src_a, dst_a, val_a)
        start_gather(src_a, rows_a, semg_a)
        wait_scatter(rows_b, dst_b, sems_b)
        load_idx(k0 + 3, src_b, dst_b, val_b)
        start_gather(src_b, rows_b, semg_b)
        return carry

    lax.fori_loop(0, NCH // 2 - 1, pair_body, 0)

    wait_gather(src_a, rows_a, semg_a)
    scale(rows_a, val_a)
    start_scatter(rows_a, dst_a, sems_a)
    wait_gather(src_b, rows_b, semg_b)
    scale(rows_b, val_b)
    start_scatter(rows_b, dst_b, sems_b)
    wait_scatter(rows_a, dst_a, sems_a)
    wait_scatter(rows_b, dst_b, sems_b)
    plsc.subcore_barrier()

    pltpu.sync_copy(ab_hbm, ab_v)
    a1 = ab_v[0]
    a2 = ab_v[1]

    def fin_blk(b, carry):
        r0 = s * RPT + b * FB
        pltpu.sync_copy(acc_sh.at[pl.ds(r0, FB)], obuf)
        pltpu.sync_copy(x_hbm.at[pl.ds(r0, FB), pl.ds(c * DH, DH)], xbuf)

        def fin_row(r, carry2):
            for g in range(DH // 16):
                ov = obuf[r, pl.ds(g * 16, 16)]
                xv = xbuf[r, pl.ds(g * 16, 16)]
                obuf[r, pl.ds(g * 16, 16)] = a1 * ov + a2 * xv
            return carry2

        lax.fori_loop(0, FB, fin_row, 0)
        pltpu.sync_copy(obuf, out_hbm.at[pl.ds(r0, FB), pl.ds(c * DH, DH)])
        return carry

    lax.fori_loop(0, RPT // FB, fin_blk, 0)


@jax.jit
def _sc_call(inp2, dst, src, val, x, ab):
    mesh = plsc.VectorSubcoreMesh(core_axis_name="c", subcore_axis_name="s")
    f = functools.partial(
        pl.kernel,
        out_type=jax.ShapeDtypeStruct((N, D), jnp.float32),
        mesh=mesh,
        compiler_params=pltpu.CompilerParams(
            use_tc_tiling_on_sc=False, needs_layout_passes=False),
        scratch_types=[
            pltpu.VMEM_SHARED((N, DH), jnp.float32),
            pltpu.VMEM((C,), jnp.int32),
            pltpu.VMEM((C,), jnp.int32),
            pltpu.VMEM((C,), jnp.int32),
            pltpu.VMEM((C,), jnp.int32),
            pltpu.VMEM((C,), jnp.float32),
            pltpu.VMEM((C,), jnp.float32),
            pltpu.VMEM((C, DH), jnp.float32),
            pltpu.VMEM((C, DH), jnp.float32),
            pltpu.VMEM((ZR, DH), jnp.float32),
            pltpu.VMEM((FB, DH), jnp.float32),
            pltpu.VMEM((FB, DH), jnp.float32),
            pltpu.VMEM((2, 16), jnp.float32),
            pltpu.SemaphoreType.DMA,
            pltpu.SemaphoreType.DMA,
            pltpu.SemaphoreType.DMA,
            pltpu.SemaphoreType.DMA,
        ],
    )(_sc_body)
    return f(inp2, dst, src, val, x, ab)


def kernel(inp, adj_indices, adj_values, x, alpha1, alpha2):
    inp2 = inp.reshape(2 * N, DH)
    dst = adj_indices[0]
    src = adj_indices[1]
    ab = jnp.stack([jnp.full((16,), alpha1[0], jnp.float32),
                    jnp.full((16,), alpha2[0], jnp.float32)])
    return _sc_call(inp2, dst, src, adj_values, x, ab)


# speedup vs baseline: 10.1570x; 1.0879x over previous
import functools

import jax
import jax.numpy as jnp
from jax import lax
from jax.experimental import pallas as pl
from jax.experimental.pallas import tpu as pltpu
from jax.experimental.pallas import tpu_sc as plsc

N = 10000
E = 320000
D = 128
DH = D // 2

NC = 2
NS = 16

EPT = E // NS
C = 400
NCH = EPT // C
NCHP = NCH + 2
RPT = N // NS
FB = 125


def _sc_body(inp2_hbm, ei_hbm, x_hbm, ab_hbm, out_hbm,
             acc_sh, ebuf_a, ebuf_b, src_a, src_b, dst_a, dst_b, val_a, val_b,
             rows_a, rows_b, obuf, xbuf, ab_v,
             semi_a, semi_b, semg_a, semg_b, sems_a, sems_b):
    c = lax.axis_index("c")
    s = lax.axis_index("s")

    @plsc.parallel_loop(0, FB, unroll=4)
    def _(r):
        for g in range(DH // 16):
            obuf[r, pl.ds(g * 16, 16)] = jnp.zeros((16,), jnp.float32)

    def zero_blk(b, carry):
        pltpu.sync_copy(obuf, acc_sh.at[pl.ds(s * RPT + b * FB, FB)])
        return carry

    lax.fori_loop(0, RPT // FB, zero_blk, 0)
    plsc.subcore_barrier()

    def start_idx(k, ebuf, sem):
        pltpu.async_copy(ei_hbm.at[s * NCHP + k], ebuf, sem)

    def wait_idx(k, ebuf, sem):
        pltpu.make_async_copy(ei_hbm.at[s * NCHP + k], ebuf, sem).wait()

    def transform(ebuf, sbuf, dbuf, vbuf):
        @plsc.parallel_loop(0, C // 16, unroll=4)
        def _(g):
            sl = pl.ds(g * 16, 16)
            dbuf[sl] = ebuf[0, sl]
            sbuf[sl] = ebuf[1, sl] + c
            vbuf[sl] = plsc.bitcast(ebuf[2, sl], jnp.float32)

    def start_gather(sbuf, rbuf, sem):
        pltpu.async_copy(inp2_hbm.at[sbuf], rbuf, sem)

    def wait_gather(sbuf, rbuf, sem):
        pltpu.make_async_copy(inp2_hbm.at[sbuf], rbuf, sem).wait()

    def scale(rbuf, vbuf):
        @plsc.parallel_loop(0, C, unroll=8)
        def _(e):
            vs = plsc.load_gather(vbuf, [jnp.full((16,), e, jnp.int32)])
            for g in range(DH // 16):
                rbuf[e, pl.ds(g * 16, 16)] = rbuf[e, pl.ds(g * 16, 16)] * vs

    def start_scatter(rbuf, dbuf, sem):
        pltpu.async_copy(rbuf, acc_sh.at[dbuf], sem, add=True)

    def wait_scatter(rbuf, dbuf, sem):
        pltpu.make_async_copy(rbuf, acc_sh.at[dbuf], sem).wait()

    start_idx(0, ebuf_a, semi_a)
    start_idx(1, ebuf_b, semi_b)
    wait_idx(0, ebuf_a, semi_a)
    transform(ebuf_a, src_a, dst_a, val_a)
    start_idx(2, ebuf_a, semi_a)
    start_gather(src_a, rows_a, semg_a)
    wait_idx(1, ebuf_b, semi_b)
    transform(ebuf_b, src_b, dst_b, val_b)
    start_idx(3, ebuf_b, semi_b)
    start_gather(src_b, rows_b, semg_b)

    def pair_body(g, carry):
        k0 = 2 * g
        wait_gather(src_a, rows_a, semg_a)
        scale(rows_a, val_a)
        start_scatter(rows_a, dst_a, sems_a)
        wait_gather(src_b, rows_b, semg_b)
        scale(rows_b, val_b)
        start_scatter(rows_b, dst_b, sems_b)
        wait_idx(k0 + 2, ebuf_a, semi_a)
        wait_scatter(rows_a, dst_a, sems_a)
        transform(ebuf_a, src_a, dst_a, val_a)
        start_idx(k0 + 4, ebuf_a, semi_a)
        start_gather(src_a, rows_a, semg_a)
        wait_idx(k0 + 3, ebuf_b, semi_b)
        wait_scatter(rows_b, dst_b, sems_b)
        transform(ebuf_b, src_b, dst_b, val_b)
        start_idx(k0 + 5, ebuf_b, semi_b)
        start_gather(src_b, rows_b, semg_b)
        return carry

    lax.fori_loop(0, NCH // 2 - 1, pair_body, 0)

    wait_gather(src_a, rows_a, semg_a)
    scale(rows_a, val_a)
    start_scatter(rows_a, dst_a, sems_a)
    wait_gather(src_b, rows_b, semg_b)
    scale(rows_b, val_b)
    start_scatter(rows_b, dst_b, sems_b)
    wait_idx(NCH, ebuf_a, semi_a)
    wait_idx(NCH + 1, ebuf_b, semi_b)
    wait_scatter(rows_a, dst_a, sems_a)
    wait_scatter(rows_b, dst_b, sems_b)
    plsc.subcore_barrier()

    pltpu.sync_copy(ab_hbm, ab_v)
    a1 = ab_v[0]
    a2 = ab_v[1]

    def fin_blk(b, carry):
        r0 = s * RPT + b * FB
        pltpu.sync_copy(acc_sh.at[pl.ds(r0, FB)], obuf)
        pltpu.sync_copy(x_hbm.at[pl.ds(r0, FB), pl.ds(c * DH, DH)], xbuf)

        @plsc.parallel_loop(0, FB, unroll=4)
        def _(r):
            for g in range(DH // 16):
                ov = obuf[r, pl.ds(g * 16, 16)]
                xv = xbuf[r, pl.ds(g * 16, 16)]
                obuf[r, pl.ds(g * 16, 16)] = a1 * ov + a2 * xv

        pltpu.sync_copy(obuf, out_hbm.at[pl.ds(r0, FB), pl.ds(c * DH, DH)])
        return carry

    lax.fori_loop(0, RPT // FB, fin_blk, 0)


@jax.jit
def _sc_call(inp2, ei, x, ab):
    mesh = plsc.VectorSubcoreMesh(core_axis_name="c", subcore_axis_name="s")
    f = functools.partial(
        pl.kernel,
        out_type=jax.ShapeDtypeStruct((N, D), jnp.float32),
        mesh=mesh,
        compiler_params=pltpu.CompilerParams(
            use_tc_tiling_on_sc=False, needs_layout_passes=False),
        scratch_types=[
            pltpu.VMEM_SHARED((N, DH), jnp.float32),
            pltpu.VMEM((3, C), jnp.int32),
            pltpu.VMEM((3, C), jnp.int32),
            pltpu.VMEM((C,), jnp.int32),
            pltpu.VMEM((C,), jnp.int32),
            pltpu.VMEM((C,), jnp.int32),
            pltpu.VMEM((C,), jnp.int32),
            pltpu.VMEM((C,), jnp.float32),
            pltpu.VMEM((C,), jnp.float32),
            pltpu.VMEM((C, DH), jnp.float32),
            pltpu.VMEM((C, DH), jnp.float32),
            pltpu.VMEM((FB, DH), jnp.float32),
            pltpu.VMEM((FB, DH), jnp.float32),
            pltpu.VMEM((2, 16), jnp.float32),
            pltpu.SemaphoreType.DMA,
            pltpu.SemaphoreType.DMA,
            pltpu.SemaphoreType.DMA,
            pltpu.SemaphoreType.DMA,
            pltpu.SemaphoreType.DMA,
            pltpu.SemaphoreType.DMA,
        ],
    )(_sc_body)
    return f(inp2, ei, x, ab)


def kernel(inp, adj_indices, adj_values, x, alpha1, alpha2):
    inp2 = inp.reshape(2 * N, DH)
    dst = adj_indices[0]
    src2 = adj_indices[1] * 2
    valb = lax.bitcast_convert_type(adj_values, jnp.int32)
    ei = jnp.stack([dst.reshape(NS, NCH, C), src2.reshape(NS, NCH, C),
                    valb.reshape(NS, NCH, C)], axis=2)
    pad = jnp.zeros((NS, 2, 3, C), jnp.int32)
    ei = jnp.concatenate([ei, pad], axis=1).reshape(NS * NCHP, 3, C)
    ab = jnp.stack([jnp.full((16,), alpha1[0], jnp.float32),
                    jnp.full((16,), alpha2[0], jnp.float32)])
    return _sc_call(inp2, ei, x, ab)
